# Initial kernel scaffold; baseline (speedup 1.0000x reference)
import jax
import jax.numpy as jnp
from jax.experimental import pallas as pl


def kernel(z, pos, batch, emb, mlp_w1, mlp_b1, mlp_w2, mlp_b2, lin1_w, lin2_w, lin2_b, lin_w, lin_b):
    raise NotImplementedError("write your pallas kernel here")


# speedup vs baseline: 20.0589x; 20.0589x over previous
import functools

import jax
import jax.numpy as jnp
import numpy as np
from jax.experimental import pallas as pl
from jax.experimental.pallas import tpu as pltpu

HID = 128
NINT = 3
RCUT = 5.0
NGAUSS = 50
NA = 8192
NG = 256
ZMAX = 100

BD = 32
CH = 128
RB = 1024

_GAP = RCUT / (NGAUSS - 1)
_COEFF = -0.5 / (_GAP * _GAP)
_LOG2 = float(np.log(2.0))


def _ssp(x):
    return jnp.maximum(x, 0.0) + jnp.log1p(jnp.exp(-jnp.abs(x))) - _LOG2


def _embed_kernel(zc_ref, emb_ref, l1_ref, h_ref, xs_ref):
    zc = zc_ref[...]
    zid = jax.lax.broadcasted_iota(jnp.int32, (RB, ZMAX), 1).astype(jnp.float32)
    oh = (zc == zid).astype(jnp.float32)
    h = jnp.dot(oh, emb_ref[...], preferred_element_type=jnp.float32)
    h_ref[...] = h
    xs_ref[...] = jnp.dot(h, l1_ref[...], preferred_element_type=jnp.float32)


def _msg_kernel(lo_ref, nch_ref, meta_ref, xs_ref, w1_ref, b1_ref,
                w2_ref, b2_ref, out_ref):
    b = pl.program_id(0)
    j0 = b * BD
    dstf = meta_ref[pl.ds(j0, BD), :]
    lo = lo_ref[b]
    nch = nch_ref[b]
    offs = jax.lax.broadcasted_iota(jnp.int32, (1, NGAUSS), 1).astype(jnp.float32) * _GAP

    dstA = jnp.broadcast_to(jnp.reshape(dstf, (BD, 1, 8)),
                            (BD, CH, 8)).reshape(BD * CH, 8)

    w1 = w1_ref[...]
    b1 = b1_ref[...]
    w2 = w2_ref[...]
    b2 = b2_ref[...]

    def body(k, acc):
        start = lo + k * CH
        cs = jnp.minimum(start, NA - CH)
        src = meta_ref[pl.ds(cs, CH), :]
        xs_c = xs_ref[pl.ds(cs, CH), :]
        srcB = jnp.broadcast_to(jnp.reshape(src, (1, CH, 8)),
                                (BD, CH, 8)).reshape(BD * CH, 8)
        d = dstA - srcB
        dist2 = d[:, 0:1] * d[:, 0:1] + d[:, 1:2] * d[:, 1:2] + d[:, 2:3] * d[:, 2:3]
        beq = dstA[:, 3:4] == srcB[:, 3:4]
        ine = dstA[:, 4:5] != srcB[:, 4:5]
        startf = start.astype(jnp.float32)
        sidx = srcB[:, 4:5]
        inwin = (sidx >= startf) & (sidx < startf + CH)
        dd = jnp.sqrt(jnp.maximum(dist2, 1e-12))
        cosc = 0.5 * (jnp.cos(dd * (np.pi / RCUT)) + 1.0)
        ok = beq & ine & (dist2 < RCUT * RCUT) & inwin
        scale = jnp.where(ok, cosc, 0.0)
        rbf = jnp.exp(_COEFF * (dd - offs) ** 2)
        pre = jnp.dot(rbf, w1, preferred_element_type=jnp.float32) + b1
        act = _ssp(pre)
        W = jnp.dot(act, w2, preferred_element_type=jnp.float32) + b2
        xs_b = jnp.broadcast_to(jnp.reshape(xs_c, (1, CH, HID)),
                                (BD, CH, HID)).reshape(BD * CH, HID)
        msg = (W * scale) * xs_b
        return acc + jnp.sum(jnp.reshape(msg, (BD, CH, HID)), axis=1)

    acc0 = jnp.zeros((BD, HID), jnp.float32)
    out_ref[...] = jax.lax.fori_loop(0, nch, body, acc0)


def _upd_kernel(h_ref, agg_ref, l2w_ref, l2b_ref, lw_ref, lb_ref, nw_ref,
                h_out_ref, xs_out_ref):
    v = jnp.dot(agg_ref[...], l2w_ref[...],
                preferred_element_type=jnp.float32) + l2b_ref[...]
    a = _ssp(v)
    hn = h_ref[...] + jnp.dot(a, lw_ref[...],
                              preferred_element_type=jnp.float32) + lb_ref[...]
    h_out_ref[...] = hn
    xs_out_ref[...] = jnp.dot(hn, nw_ref[...], preferred_element_type=jnp.float32)


def _pool_kernel(bc_ref, h_ref, out_ref):
    bc = bc_ref[...]
    gid = jax.lax.broadcasted_iota(jnp.int32, (NA, NG), 1).astype(jnp.float32)
    oh = (bc == gid).astype(jnp.float32)
    sums = jax.lax.dot_general(oh, h_ref[...], (((0,), (0,)), ((), ())),
                               preferred_element_type=jnp.float32)
    ones8 = jnp.ones((NA, 8), jnp.float32)
    cnt = jax.lax.dot_general(oh, ones8, (((0,), (0,)), ((), ())),
                              preferred_element_type=jnp.float32)[:, 0:1]
    out_ref[...] = sums / jnp.maximum(cnt, 1.0)


def kernel(z, pos, batch, emb, mlp_w1, mlp_b1, mlp_w2, mlp_b2,
           lin1_w, lin2_w, lin2_b, lin_w, lin_b):
    nb = NA // BD
    batch_i = batch.astype(jnp.int32)

    idxf = jnp.arange(NA, dtype=jnp.float32)[:, None]
    meta = jnp.concatenate(
        [pos.astype(jnp.float32), batch_i.astype(jnp.float32)[:, None], idxf,
         jnp.zeros((NA, 3), jnp.float32)], axis=1)

    starts = jnp.searchsorted(batch_i, jnp.arange(NG + 1, dtype=jnp.int32),
                              side="left").astype(jnp.int32)
    bb = batch_i.reshape(nb, BD)
    lo = starts[bb[:, 0]]
    hi = starts[bb[:, -1] + 1]
    lo8 = (lo // 8) * 8
    nch = (hi - lo8 + CH - 1) // CH

    zc = z.astype(jnp.float32)[:, None]
    bc = batch_i.astype(jnp.float32)[:, None]

    h, xs = pl.pallas_call(
        _embed_kernel,
        grid=(NA // RB,),
        in_specs=[pl.BlockSpec((RB, 1), lambda r: (r, 0)),
                  pl.BlockSpec((ZMAX, HID), lambda r: (0, 0)),
                  pl.BlockSpec((HID, HID), lambda r: (0, 0))],
        out_specs=[pl.BlockSpec((RB, HID), lambda r: (r, 0)),
                   pl.BlockSpec((RB, HID), lambda r: (r, 0))],
        out_shape=[jax.ShapeDtypeStruct((NA, HID), jnp.float32),
                   jax.ShapeDtypeStruct((NA, HID), jnp.float32)],
    )(zc, emb, lin1_w[0])

    msg_call = pl.pallas_call(
        _msg_kernel,
        grid_spec=pltpu.PrefetchScalarGridSpec(
            num_scalar_prefetch=2,
            grid=(nb,),
            in_specs=[
                pl.BlockSpec((NA, 8), lambda b, lo_r, nch_r: (0, 0)),
                pl.BlockSpec((NA, HID), lambda b, lo_r, nch_r: (0, 0)),
                pl.BlockSpec((NGAUSS, HID), lambda b, lo_r, nch_r: (0, 0)),
                pl.BlockSpec((1, HID), lambda b, lo_r, nch_r: (0, 0)),
                pl.BlockSpec((HID, HID), lambda b, lo_r, nch_r: (0, 0)),
                pl.BlockSpec((1, HID), lambda b, lo_r, nch_r: (0, 0)),
            ],
            out_specs=pl.BlockSpec((BD, HID), lambda b, lo_r, nch_r: (b, 0)),
        ),
        out_shape=jax.ShapeDtypeStruct((NA, HID), jnp.float32),
    )

    upd_call = pl.pallas_call(
        _upd_kernel,
        grid=(NA // RB,),
        in_specs=[pl.BlockSpec((RB, HID), lambda r: (r, 0)),
                  pl.BlockSpec((RB, HID), lambda r: (r, 0)),
                  pl.BlockSpec((HID, HID), lambda r: (0, 0)),
                  pl.BlockSpec((1, HID), lambda r: (0, 0)),
                  pl.BlockSpec((HID, HID), lambda r: (0, 0)),
                  pl.BlockSpec((1, HID), lambda r: (0, 0)),
                  pl.BlockSpec((HID, HID), lambda r: (0, 0))],
        out_specs=[pl.BlockSpec((RB, HID), lambda r: (r, 0)),
                   pl.BlockSpec((RB, HID), lambda r: (r, 0))],
        out_shape=[jax.ShapeDtypeStruct((NA, HID), jnp.float32),
                   jax.ShapeDtypeStruct((NA, HID), jnp.float32)],
    )

    for i in range(NINT):
        agg = msg_call(lo8, nch, meta, xs, mlp_w1[i], mlp_b1[i][None, :],
                       mlp_w2[i], mlp_b2[i][None, :])
        nxt = lin1_w[i + 1] if i + 1 < NINT else lin1_w[0]
        h, xs = upd_call(h, agg, lin2_w[i], lin2_b[i][None, :],
                         lin_w[i], lin_b[i][None, :], nxt)

    return pl.pallas_call(
        _pool_kernel,
        in_specs=[pl.BlockSpec((NA, 1), lambda: (0, 0)),
                  pl.BlockSpec((NA, HID), lambda: (0, 0))],
        out_specs=pl.BlockSpec((NG, HID), lambda: (0, 0)),
        out_shape=jax.ShapeDtypeStruct((NG, HID), jnp.float32),
    )(bc, h)
